# baseline (speedup 1.0000x reference)
import functools

import jax
import jax.numpy as jnp
from jax import lax
from jax.experimental import pallas as pl
from jax.experimental.pallas import tpu as pltpu
from jax.experimental.pallas import tpu_sc as plsc

H = 64
IDX_CHUNK = 128
BB = 256


def _sc_gather(table, idx):
    n = idx.shape[0]
    info = plsc.get_sparse_core_info()
    nw = info.num_cores * info.num_subcores
    assert n % nw == 0
    b_per_w = n // nw
    assert b_per_w % 8 == 0
    sizes = []
    left = b_per_w
    while left > 0:
        s = min(IDX_CHUNK, left)
        sizes.append(s)
        left -= s

    mesh = plsc.VectorSubcoreMesh(core_axis_name="c", subcore_axis_name="s")

    @functools.partial(
        pl.kernel,
        out_type=jax.ShapeDtypeStruct((n, H), jnp.float32),
        mesh=mesh,
        scratch_types=[
            pltpu.VMEM((b_per_w,), jnp.int32),
            pltpu.VMEM((b_per_w, H), jnp.float32),
            pltpu.SemaphoreType.DMA,
        ],
        compiler_params=pltpu.CompilerParams(use_tc_tiling_on_sc=False),
    )
    def k(table_hbm, idx_hbm, out_hbm, idx_v, rows_v, sem):
        wid = lax.axis_index("s") * info.num_cores + lax.axis_index("c")
        base = wid * b_per_w
        pltpu.sync_copy(idx_hbm.at[pl.ds(base, b_per_w)], idx_v)
        copies = []
        off = 0
        for s in sizes:
            copies.append(
                pltpu.async_copy(
                    table_hbm.at[idx_v.at[pl.ds(off, s)]],
                    rows_v.at[pl.ds(off, s)],
                    sem,
                )
            )
            off += s
        for c in copies:
            c.wait()
        pltpu.sync_copy(rows_v, out_hbm.at[pl.ds(base, b_per_w)])

    return k(table, idx)


def _lstm_tc(emb, ctx, h0f, c0f, h0b, c0b, Wf, bf, Wb, bb):
    B, T, _ = emb.shape
    nb = B // BB

    def body(emb_r, ctx_r, h0f_r, c0f_r, h0b_r, c0b_r,
             wf_r, bf_r, wb_r, bb_r,
             out_r, hf_o, cf_o, hb_o, cb_o):
        wf = wf_r[:]
        bfv = bf_r[:]
        wb = wb_r[:]
        bbv = bb_r[:]

        def step(emb_t, ctx_t, h, c, w, b):
            x = jnp.concatenate([emb_t, ctx_t, h], axis=-1)
            g = jnp.dot(x, w, preferred_element_type=jnp.float32) + b
            s = jax.nn.sigmoid(g)
            i = s[:, 0 * H:1 * H]
            f = s[:, 1 * H:2 * H]
            gg = 2.0 * s[:, 2 * H:3 * H] - 1.0
            o = s[:, 3 * H:4 * H]
            c2 = f * c + i * gg
            h2 = o * jnp.tanh(c2)
            return h2, c2

        def loop(t, carry):
            hf, cf, hb, cb = carry
            rt = T - 1 - t
            hf, cf = step(emb_r[:, t, :], ctx_r[:, t, :], hf, cf, wf, bfv)
            out_r[:, t, 0:H] = hf
            hb, cb = step(emb_r[:, rt, :], ctx_r[:, rt, :], hb, cb, wb, bbv)
            out_r[:, rt, H:2 * H] = hb
            return hf, cf, hb, cb

        init = (h0f_r[:], c0f_r[:], h0b_r[:], c0b_r[:])
        hf, cf, hb, cb = lax.fori_loop(0, T, loop, init, unroll=2)
        hf_o[:] = hf
        cf_o[:] = cf
        hb_o[:] = hb
        cb_o[:] = cb

    bblk = lambda w: pl.BlockSpec((BB, T, w), lambda i: (i, 0, 0))
    bvec = lambda: pl.BlockSpec((BB, H), lambda i: (i, 0))
    full = lambda shape: pl.BlockSpec(shape, lambda i: (0,) * len(shape))

    in_specs = [
        bblk(H), bblk(2 * H),
        bvec(), bvec(), bvec(), bvec(),
        full((4 * H, 4 * H)), full((1, 4 * H)),
        full((4 * H, 4 * H)), full((1, 4 * H)),
    ]
    out_specs = [
        bblk(2 * H),
        bvec(), bvec(), bvec(), bvec(),
    ]
    out_shape = [
        jax.ShapeDtypeStruct((B, T, 2 * H), jnp.float32),
        jax.ShapeDtypeStruct((B, H), jnp.float32),
        jax.ShapeDtypeStruct((B, H), jnp.float32),
        jax.ShapeDtypeStruct((B, H), jnp.float32),
        jax.ShapeDtypeStruct((B, H), jnp.float32),
    ]
    return pl.pallas_call(
        body,
        grid=(nb,),
        in_specs=in_specs,
        out_specs=out_specs,
        out_shape=out_shape,
    )(emb, ctx, h0f, c0f, h0b, c0b, Wf, bf, Wb, bb)


def kernel(inputs, context, decoder_hidden_state, decoder_cell_state, table,
           Wih_f, Whh_f, bih_f, bhh_f, Wih_b, Whh_b, bih_b, bhh_b):
    B, T = inputs.shape

    idx = inputs.reshape(-1).astype(jnp.int32)
    emb = _sc_gather(table, idx).reshape(B, T, H)

    gate_scale = jnp.concatenate(
        [jnp.ones((2 * H,), jnp.float32), jnp.full((H,), 2.0, jnp.float32),
         jnp.ones((H,), jnp.float32)])
    Wf = jnp.concatenate([Wih_f.T, Whh_f.T], axis=0) * gate_scale
    Wb = jnp.concatenate([Wih_b.T, Whh_b.T], axis=0) * gate_scale
    bf = ((bih_f + bhh_f) * gate_scale).reshape(1, -1)
    bb = ((bih_b + bhh_b) * gate_scale).reshape(1, -1)

    out, hf, cf, hb, cb = _lstm_tc(
        emb, context,
        decoder_hidden_state[0], decoder_cell_state[0],
        decoder_hidden_state[1], decoder_cell_state[1],
        Wf, bf, Wb, bb)

    h_n = jnp.stack([hf, hb], axis=0)
    c_n = jnp.stack([cf, cb], axis=0)
    return out, h_n, c_n


# speedup vs baseline: 1.2059x; 1.2059x over previous
import functools

import jax
import jax.numpy as jnp
from jax import lax
from jax.experimental import pallas as pl
from jax.experimental.pallas import tpu as pltpu
from jax.experimental.pallas import tpu_sc as plsc

H = 64
IDX_CHUNK = 128
TB = 5


def _sc_gather(table, idx):
    n = idx.shape[0]
    info = plsc.get_sparse_core_info()
    nw = info.num_cores * info.num_subcores
    assert n % nw == 0
    b_per_w = n // nw
    assert b_per_w % 8 == 0
    sizes = []
    left = b_per_w
    while left > 0:
        s = min(IDX_CHUNK, left)
        sizes.append(s)
        left -= s

    mesh = plsc.VectorSubcoreMesh(core_axis_name="c", subcore_axis_name="s")

    @functools.partial(
        pl.kernel,
        out_type=jax.ShapeDtypeStruct((n, H), jnp.float32),
        mesh=mesh,
        scratch_types=[
            pltpu.VMEM((b_per_w,), jnp.int32),
            pltpu.VMEM((b_per_w, H), jnp.float32),
            pltpu.SemaphoreType.DMA,
        ],
        compiler_params=pltpu.CompilerParams(use_tc_tiling_on_sc=False),
    )
    def k(table_hbm, idx_hbm, out_hbm, idx_v, rows_v, sem):
        wid = lax.axis_index("s") * info.num_cores + lax.axis_index("c")
        base = wid * b_per_w
        pltpu.sync_copy(idx_hbm.at[pl.ds(base, b_per_w)], idx_v)
        copies = []
        off = 0
        for s in sizes:
            copies.append(
                pltpu.async_copy(
                    table_hbm.at[idx_v.at[pl.ds(off, s)]],
                    rows_v.at[pl.ds(off, s)],
                    sem,
                )
            )
            off += s
        for c in copies:
            c.wait()
        pltpu.sync_copy(rows_v, out_hbm.at[pl.ds(base, b_per_w)])

    return k(table, idx)


def _lstm_tc(emb_tm, ctx_tm, h0f, c0f, h0b, c0b, Wf, bf, Wb, bb):
    T, B, _ = emb_tm.shape
    nt = T // TB
    assert nt * TB == T

    def body(emb_f, ctx_f, emb_b, ctx_b, h0f_r, c0f_r, h0b_r, c0b_r,
             wf_r, bf_r, wb_r, bb_r,
             out_f, out_b, hf_o, cf_o, hb_o, cb_o,
             hf_s, cf_s, hb_s, cb_s):
        i = pl.program_id(0)

        @pl.when(i == 0)
        def _():
            hf_s[:] = h0f_r[:]
            cf_s[:] = c0f_r[:]
            hb_s[:] = h0b_r[:]
            cb_s[:] = c0b_r[:]

        def step(emb, ctx_t, h, c, w, b):
            x = jnp.concatenate([emb, ctx_t, h], axis=-1)
            g = jnp.dot(x, w, preferred_element_type=jnp.float32) + b
            s = jax.nn.sigmoid(g)
            gi = s[:, 0 * H:1 * H]
            gf = s[:, 1 * H:2 * H]
            gg = 2.0 * s[:, 2 * H:3 * H] - 1.0
            go = s[:, 3 * H:4 * H]
            c2 = gf * c + gi * gg
            h2 = go * jnp.tanh(c2)
            return h2, c2

        hf, cf = hf_s[:], cf_s[:]
        hb, cb = hb_s[:], cb_s[:]
        for k in range(TB):
            hf, cf = step(emb_f[k], ctx_f[k], hf, cf, wf_r[:], bf_r[:])
            out_f[k] = hf
            rk = TB - 1 - k
            hb, cb = step(emb_b[rk], ctx_b[rk], hb, cb, wb_r[:], bb_r[:])
            out_b[rk] = hb
        hf_s[:] = hf
        cf_s[:] = cf
        hb_s[:] = hb
        cb_s[:] = cb

        @pl.when(i == nt - 1)
        def _():
            hf_o[:] = hf
            cf_o[:] = cf
            hb_o[:] = hb
            cb_o[:] = cb

    full = lambda shape: pl.BlockSpec(shape, lambda i: (0,) * len(shape))
    tspec = lambda w: pl.BlockSpec((TB, B, w), lambda i: (i, 0, 0))
    rspec = lambda w: pl.BlockSpec((TB, B, w), lambda i: (nt - 1 - i, 0, 0))

    in_specs = [
        tspec(H), tspec(2 * H), rspec(H), rspec(2 * H),
        full((B, H)), full((B, H)), full((B, H)), full((B, H)),
        full((4 * H, 4 * H)), full((1, 4 * H)),
        full((4 * H, 4 * H)), full((1, 4 * H)),
    ]
    out_specs = [
        tspec(H), rspec(H),
        full((B, H)), full((B, H)), full((B, H)), full((B, H)),
    ]
    out_shape = [
        jax.ShapeDtypeStruct((T, B, H), jnp.float32),
        jax.ShapeDtypeStruct((T, B, H), jnp.float32),
        jax.ShapeDtypeStruct((B, H), jnp.float32),
        jax.ShapeDtypeStruct((B, H), jnp.float32),
        jax.ShapeDtypeStruct((B, H), jnp.float32),
        jax.ShapeDtypeStruct((B, H), jnp.float32),
    ]
    scratch = [pltpu.VMEM((B, H), jnp.float32)] * 4
    return pl.pallas_call(
        body,
        grid=(nt,),
        in_specs=in_specs,
        out_specs=out_specs,
        out_shape=out_shape,
        scratch_shapes=scratch,
    )(emb_tm, ctx_tm, emb_tm, ctx_tm, h0f, c0f, h0b, c0b, Wf, bf, Wb, bb)


def kernel(inputs, context, decoder_hidden_state, decoder_cell_state, table,
           Wih_f, Whh_f, bih_f, bhh_f, Wih_b, Whh_b, bih_b, bhh_b):
    B, T = inputs.shape

    idx_tm = jnp.transpose(inputs).reshape(-1).astype(jnp.int32)
    emb_tm = _sc_gather(table, idx_tm).reshape(T, B, H)

    ctx_tm = jnp.transpose(context, (1, 0, 2))

    gate_scale = jnp.concatenate(
        [jnp.ones((2 * H,), jnp.float32), jnp.full((H,), 2.0, jnp.float32),
         jnp.ones((H,), jnp.float32)])
    Wf = jnp.concatenate([Wih_f.T, Whh_f.T], axis=0) * gate_scale
    Wb = jnp.concatenate([Wih_b.T, Whh_b.T], axis=0) * gate_scale
    bf = ((bih_f + bhh_f) * gate_scale).reshape(1, -1)
    bb = ((bih_b + bhh_b) * gate_scale).reshape(1, -1)

    ys_f, ys_b, hf, cf, hb, cb = _lstm_tc(
        emb_tm, ctx_tm,
        decoder_hidden_state[0], decoder_cell_state[0],
        decoder_hidden_state[1], decoder_cell_state[1],
        Wf, bf, Wb, bb)

    out = jnp.transpose(jnp.concatenate([ys_f, ys_b], axis=-1), (1, 0, 2))
    h_n = jnp.stack([hf, hb], axis=0)
    c_n = jnp.stack([cf, cb], axis=0)
    return out, h_n, c_n
